# baseline (speedup 1.0000x reference)
import functools

import jax
import jax.numpy as jnp
from jax import lax
from jax.experimental import pallas as pl
from jax.experimental.pallas import tpu as pltpu
from jax.experimental.pallas import tpu_sc as plsc

_N_SKELS = 100000
_N_TRAIN = 80000
_N_TOKENS = 8
_CODE_DIM = 64
_BATCH = 16384

_NC = 2
_NS = 16
_NW = _NC * _NS
_B_PER_W = _BATCH // _NW
_G = 16
_N_GROUPS = _B_PER_W // _G
_LANES = 16


def _make_kernel():
    mesh = plsc.VectorSubcoreMesh(core_axis_name="c", subcore_axis_name="s")
    row3 = (_N_TOKENS, _CODE_DIM)

    @functools.partial(
        pl.kernel,
        out_type=jax.ShapeDtypeStruct((_BATCH,) + row3, jnp.float32),
        mesh=mesh,
        compiler_params=pltpu.CompilerParams(needs_layout_passes=False),
        scratch_types=[
            pltpu.VMEM((_B_PER_W,), jnp.int32),
            pltpu.VMEM((_G,) + row3, jnp.float32),
            pltpu.VMEM((_G,) + row3, jnp.float32),
            pltpu.SemaphoreType.DMA,
            pltpu.SemaphoreType.DMA,
        ],
    )
    def k(idx_hbm, zm_hbm, off_hbm, out_hbm, idx_v, mean_v, off_v, sem_m, sem_o):
        wid = lax.axis_index("s") * _NC + lax.axis_index("c")
        base = wid * _B_PER_W
        pltpu.sync_copy(idx_hbm.at[pl.ds(base, _B_PER_W)], idx_v)

        def group_body(g, carry):
            ids16 = idx_v[pl.ds(g * _G, _G)]
            mvec = jnp.where(ids16 < _N_TRAIN, jnp.float32(1.0),
                             jnp.float32(0.0))
            cps = []
            for r in range(_G):
                rid = ids16[r]
                cps.append(pltpu.async_copy(zm_hbm.at[rid], mean_v.at[r],
                                            sem_m))
                cps.append(pltpu.async_copy(off_hbm.at[rid], off_v.at[r],
                                            sem_o))
            for cp in cps:
                cp.wait()

            for r in range(_G):
                m = mvec[r]
                for t in range(_N_TOKENS):
                    for c in range(_CODE_DIM // _LANES):
                        sl = pl.ds(c * _LANES, _LANES)
                        mean_v[r, t, sl] = (mean_v[r, t, sl]
                                            + off_v[r, t, sl] * m)

            pltpu.sync_copy(mean_v, out_hbm.at[pl.ds(base + g * _G, _G)])
            return carry

        lax.fori_loop(0, _N_GROUPS, group_body, 0, unroll=False)

    return k


_kernel_call = _make_kernel()


@jax.jit
def kernel(tgt_skel_id, z_means, offset):
    return _kernel_call(tgt_skel_id, z_means, offset)


# speedup vs baseline: 1.3199x; 1.3199x over previous
import functools

import jax
import jax.numpy as jnp
from jax import lax
from jax.experimental import pallas as pl
from jax.experimental.pallas import tpu as pltpu
from jax.experimental.pallas import tpu_sc as plsc

_N_SKELS = 100000
_N_TRAIN = 80000
_N_TOKENS = 8
_CODE_DIM = 64
_BATCH = 16384
_HALF = _BATCH // 2

_NC = 2
_NS = 16
_NW = _NC * _NS
_N_PLANES = _N_TOKENS * _CODE_DIM
_P_PER_W = _N_PLANES // _NW
_LANES = 16


def _make_kernel():
    mesh = plsc.VectorSubcoreMesh(core_axis_name="c", subcore_axis_name="s")

    @functools.partial(
        pl.kernel,
        out_type=jax.ShapeDtypeStruct((_N_TOKENS, _CODE_DIM, _BATCH),
                                      jnp.float32),
        mesh=mesh,
        compiler_params=pltpu.CompilerParams(needs_layout_passes=False),
        scratch_types=[
            pltpu.VMEM((_N_SKELS,), jnp.float32),
            pltpu.VMEM((_HALF,), jnp.int32),
            pltpu.VMEM((_BATCH,), jnp.float32),
            pltpu.SemaphoreType.DMA,
        ],
    )
    def k(idx_hbm, zm_hbm, off_hbm, out_hbm, plane_v, ids_v, out_v, sem):
        wid = lax.axis_index("s") * _NC + lax.axis_index("c")

        def plane_body(p, carry):
            pid = wid * _P_PER_W + p
            t = pid // _CODE_DIM
            c = pid % _CODE_DIM

            pltpu.sync_copy(zm_hbm.at[t, c], plane_v)

            def zm_half(h, carry2):
                pltpu.sync_copy(idx_hbm.at[pl.ds(h * _HALF, _HALF)], ids_v)

                def grp(j, carry3):
                    ids16 = ids_v[pl.ds(j * _LANES, _LANES)]
                    vals = plsc.load_gather(plane_v, [ids16])
                    out_v[pl.ds(h * _HALF + j * _LANES, _LANES)] = vals
                    return carry3

                lax.fori_loop(0, _HALF // _LANES, grp, 0, unroll=False)
                return carry2

            lax.fori_loop(0, 2, zm_half, 0, unroll=False)

            pltpu.sync_copy(off_hbm.at[t, c], plane_v)

            def off_half(h, carry2):
                pltpu.sync_copy(idx_hbm.at[pl.ds(h * _HALF, _HALF)], ids_v)

                def grp(j, carry3):
                    ids16 = ids_v[pl.ds(j * _LANES, _LANES)]
                    vals = plsc.load_gather(plane_v, [ids16])
                    mvec = jnp.where(ids16 < _N_TRAIN, jnp.float32(1.0),
                                     jnp.float32(0.0))
                    osl = pl.ds(h * _HALF + j * _LANES, _LANES)
                    out_v[osl] = out_v[osl] + vals * mvec
                    return carry3

                lax.fori_loop(0, _HALF // _LANES, grp, 0, unroll=False)
                return carry2

            lax.fori_loop(0, 2, off_half, 0, unroll=False)

            pltpu.sync_copy(out_v, out_hbm.at[t, c])
            return carry

        lax.fori_loop(0, _P_PER_W, plane_body, 0, unroll=False)

    return k


_kernel_call = _make_kernel()


@jax.jit
def kernel(tgt_skel_id, z_means, offset):
    zm_t = jnp.transpose(z_means, (1, 2, 0))
    off_t = jnp.transpose(offset, (1, 2, 0))
    out_t = _kernel_call(tgt_skel_id, zm_t, off_t)
    return jnp.transpose(out_t, (2, 0, 1))


# speedup vs baseline: 1.3803x; 1.0457x over previous
import functools

import jax
import jax.numpy as jnp
from jax import lax
from jax.experimental import pallas as pl
from jax.experimental.pallas import tpu as pltpu
from jax.experimental.pallas import tpu_sc as plsc

_N_SKELS = 100000
_N_TRAIN = 80000
_N_TOKENS = 8
_CODE_DIM = 64
_BATCH = 16384
_HALF = _BATCH // 2

_NC = 2
_NS = 16
_NW = _NC * _NS
_N_PLANES = _N_TOKENS * _CODE_DIM
_P_PER_W = _N_PLANES // _NW
_LANES = 16


def _make_kernel():
    mesh = plsc.VectorSubcoreMesh(core_axis_name="c", subcore_axis_name="s")

    @functools.partial(
        pl.kernel,
        out_type=jax.ShapeDtypeStruct((_N_TOKENS, _CODE_DIM, _BATCH),
                                      jnp.float32),
        mesh=mesh,
        compiler_params=pltpu.CompilerParams(needs_layout_passes=False),
        scratch_types=[
            pltpu.VMEM((_N_SKELS,), jnp.float32),
            pltpu.VMEM((_HALF,), jnp.int32),
            pltpu.VMEM((_BATCH,), jnp.float32),
            pltpu.SemaphoreType.DMA,
        ],
    )
    def k(idx_hbm, zm_hbm, off_hbm, out_hbm, plane_v, ids_v, out_v, sem):
        wid = lax.axis_index("s") * _NC + lax.axis_index("c")

        def plane_body(p, carry):
            pid = wid * _P_PER_W + p
            t = pid // _CODE_DIM
            c = pid % _CODE_DIM

            pltpu.sync_copy(zm_hbm.at[t, c], plane_v)
            pltpu.sync_copy(idx_hbm.at[pl.ds(0, _HALF)], ids_v)

            def zm_grp(h):
                def grp(j, carry3):
                    ids16 = ids_v[pl.ds(j * _LANES, _LANES)]
                    vals = plsc.load_gather(plane_v, [ids16])
                    out_v[pl.ds(h * _HALF + j * _LANES, _LANES)] = vals
                    return carry3
                return grp

            lax.fori_loop(0, _HALF // _LANES, zm_grp(0), 0, unroll=8)
            pltpu.sync_copy(idx_hbm.at[pl.ds(_HALF, _HALF)], ids_v)
            lax.fori_loop(0, _HALF // _LANES, zm_grp(1), 0, unroll=8)

            pltpu.sync_copy(off_hbm.at[t, c], plane_v)

            def off_grp(h):
                def grp(j, carry3):
                    ids16 = ids_v[pl.ds(j * _LANES, _LANES)]
                    vals = plsc.load_gather(plane_v, [ids16])
                    mvec = jnp.where(ids16 < _N_TRAIN, jnp.float32(1.0),
                                     jnp.float32(0.0))
                    osl = pl.ds(h * _HALF + j * _LANES, _LANES)
                    out_v[osl] = out_v[osl] + vals * mvec
                    return carry3
                return grp

            lax.fori_loop(0, _HALF // _LANES, off_grp(1), 0, unroll=8)
            pltpu.sync_copy(idx_hbm.at[pl.ds(0, _HALF)], ids_v)
            lax.fori_loop(0, _HALF // _LANES, off_grp(0), 0, unroll=8)

            pltpu.sync_copy(out_v, out_hbm.at[t, c])
            return carry

        lax.fori_loop(0, _P_PER_W, plane_body, 0, unroll=False)

    return k


_kernel_call = _make_kernel()


@jax.jit
def kernel(tgt_skel_id, z_means, offset):
    zm_t = jnp.transpose(z_means, (1, 2, 0))
    off_t = jnp.transpose(offset, (1, 2, 0))
    out_t = _kernel_call(tgt_skel_id, zm_t, off_t)
    return jnp.transpose(out_t, (2, 0, 1))


# speedup vs baseline: 2.6896x; 1.9486x over previous
import functools

import jax
import jax.numpy as jnp
from jax import lax
from jax.experimental import pallas as pl
from jax.experimental.pallas import tpu as pltpu
from jax.experimental.pallas import tpu_sc as plsc

_N_SKELS = 100000
_N_TRAIN = 80000
_N_TOKENS = 8
_CODE_DIM = 64
_BATCH = 16384
_HALF = _BATCH // 2

_NC = 2
_NS = 16
_NW = _NC * _NS
_N_PLANES = _N_TOKENS * _CODE_DIM
_P_PER_W = _N_PLANES // _NW
_LANES = 16


def _make_kernel():
    mesh = plsc.VectorSubcoreMesh(core_axis_name="c", subcore_axis_name="s")

    @functools.partial(
        pl.kernel,
        out_type=jax.ShapeDtypeStruct((_N_TOKENS, _CODE_DIM, _BATCH),
                                      jnp.float32),
        mesh=mesh,
        compiler_params=pltpu.CompilerParams(needs_layout_passes=False),
        scratch_types=[
            pltpu.VMEM((_N_SKELS,), jnp.float32),
            pltpu.VMEM((_HALF,), jnp.int32),
            pltpu.VMEM((_BATCH,), jnp.float32),
            pltpu.SemaphoreType.DMA,
        ],
    )
    def k(idx_hbm, zm_hbm, off_hbm, out_hbm, plane_v, ids_v, out_v, sem):
        wid = lax.axis_index("s") * _NC + lax.axis_index("c")

        def plane_body(p, carry):
            pid = wid * _P_PER_W + p
            t = pid // _CODE_DIM
            c = pid % _CODE_DIM

            pltpu.sync_copy(zm_hbm.at[t, c], plane_v)
            pltpu.sync_copy(idx_hbm.at[pl.ds(0, _HALF)], ids_v)

            def zm_pass(h):
                @plsc.parallel_loop(0, _HALF, step=_LANES, unroll=8)
                def _(i):
                    ids16 = ids_v[pl.ds(i, _LANES)]
                    vals = plsc.load_gather(plane_v, [ids16])
                    out_v[pl.ds(h * _HALF + i, _LANES)] = vals

            zm_pass(0)
            pltpu.sync_copy(idx_hbm.at[pl.ds(_HALF, _HALF)], ids_v)
            zm_pass(1)

            pltpu.sync_copy(off_hbm.at[t, c], plane_v)

            def off_pass(h):
                @plsc.parallel_loop(0, _HALF, step=_LANES, unroll=8)
                def _(i):
                    ids16 = ids_v[pl.ds(i, _LANES)]
                    vals = plsc.load_gather(plane_v, [ids16])
                    mvec = jnp.where(ids16 < _N_TRAIN, jnp.float32(1.0),
                                     jnp.float32(0.0))
                    osl = pl.ds(h * _HALF + i, _LANES)
                    out_v[osl] = out_v[osl] + vals * mvec

            off_pass(1)
            pltpu.sync_copy(idx_hbm.at[pl.ds(0, _HALF)], ids_v)
            off_pass(0)

            pltpu.sync_copy(out_v, out_hbm.at[t, c])
            return carry

        lax.fori_loop(0, _P_PER_W, plane_body, 0, unroll=False)

    return k


_kernel_call = _make_kernel()


@jax.jit
def kernel(tgt_skel_id, z_means, offset):
    zm_t = jnp.transpose(z_means, (1, 2, 0))
    off_t = jnp.transpose(offset, (1, 2, 0))
    out_t = _kernel_call(tgt_skel_id, zm_t, off_t)
    return jnp.transpose(out_t, (2, 0, 1))


# speedup vs baseline: 2.9424x; 1.0940x over previous
import functools

import jax
import jax.numpy as jnp
from jax import lax
from jax.experimental import pallas as pl
from jax.experimental.pallas import tpu as pltpu
from jax.experimental.pallas import tpu_sc as plsc

_N_SKELS = 100000
_N_TRAIN = 80000
_N_TOKENS = 8
_CODE_DIM = 64
_BATCH = 16384
_HALF = _BATCH // 2

_NC = 2
_NS = 16
_NW = _NC * _NS
_N_PLANES = _N_TOKENS * _CODE_DIM
_P_PER_W = _N_PLANES // _NW
_LANES = 16


def _make_kernel():
    mesh = plsc.VectorSubcoreMesh(core_axis_name="c", subcore_axis_name="s")

    @functools.partial(
        pl.kernel,
        out_type=jax.ShapeDtypeStruct((_N_TOKENS, _CODE_DIM, _BATCH),
                                      jnp.float32),
        mesh=mesh,
        compiler_params=pltpu.CompilerParams(needs_layout_passes=False),
        scratch_types=[
            pltpu.VMEM((_N_SKELS,), jnp.float32),
            pltpu.VMEM((_HALF,), jnp.int32),
            pltpu.VMEM((_BATCH,), jnp.float32),
            pltpu.SemaphoreType.DMA,
            pltpu.SemaphoreType.DMA,
        ],
    )
    def k(idx_hbm, zm_hbm, off_hbm, out_hbm, plane_v, ids_v, out_v, sem,
          sem_out):
        wid = lax.axis_index("s") * _NC + lax.axis_index("c")
        pltpu.sync_copy(idx_hbm.at[pl.ds(0, _HALF)], ids_v)

        def plane_body(p, carry):
            pid = wid * _P_PER_W + p
            t = pid // _CODE_DIM
            c = pid % _CODE_DIM

            pltpu.sync_copy(zm_hbm.at[t, c], plane_v)
            @pl.when(p > 0)
            def _drain():
                tp = (pid - 1) // _CODE_DIM
                cp_ = (pid - 1) % _CODE_DIM
                pltpu.make_async_copy(out_v, out_hbm.at[tp, cp_],
                                      sem_out).wait()

            def zm_pass(h):
                @plsc.parallel_loop(0, _HALF, step=_LANES, unroll=8)
                def _(i):
                    ids16 = ids_v[pl.ds(i, _LANES)]
                    vals = plsc.load_gather(plane_v, [ids16])
                    out_v[pl.ds(h * _HALF + i, _LANES)] = vals

            zm_pass(0)
            pltpu.sync_copy(idx_hbm.at[pl.ds(_HALF, _HALF)], ids_v)
            zm_pass(1)

            pltpu.sync_copy(off_hbm.at[t, c], plane_v)

            def off_pass(h):
                @plsc.parallel_loop(0, _HALF, step=_LANES, unroll=8)
                def _(i):
                    ids16 = ids_v[pl.ds(i, _LANES)]
                    vals = plsc.load_gather(plane_v, [ids16])
                    mvec = jnp.where(ids16 < _N_TRAIN, jnp.float32(1.0),
                                     jnp.float32(0.0))
                    osl = pl.ds(h * _HALF + i, _LANES)
                    out_v[osl] = out_v[osl] + vals * mvec

            off_pass(1)
            pltpu.sync_copy(idx_hbm.at[pl.ds(0, _HALF)], ids_v)
            off_pass(0)

            pltpu.async_copy(out_v, out_hbm.at[t, c], sem_out)
            return carry

        lax.fori_loop(0, _P_PER_W, plane_body, 0, unroll=False)
        last = wid * _P_PER_W + _P_PER_W - 1
        pltpu.make_async_copy(
            out_v, out_hbm.at[last // _CODE_DIM, last % _CODE_DIM],
            sem_out).wait()

    return k


_kernel_call = _make_kernel()


@jax.jit
def kernel(tgt_skel_id, z_means, offset):
    zm_t = jnp.transpose(z_means, (1, 2, 0))
    off_t = jnp.transpose(offset, (1, 2, 0))
    out_t = _kernel_call(tgt_skel_id, zm_t, off_t)
    return jnp.transpose(out_t, (2, 0, 1))


# speedup vs baseline: 2.9440x; 1.0005x over previous
import functools

import jax
import jax.numpy as jnp
from jax import lax
from jax.experimental import pallas as pl
from jax.experimental.pallas import tpu as pltpu
from jax.experimental.pallas import tpu_sc as plsc

_N_SKELS = 100000
_N_TRAIN = 80000
_N_TOKENS = 8
_CODE_DIM = 64
_BATCH = 16384
_HALF = _BATCH // 2

_NC = 2
_NS = 16
_NW = _NC * _NS
_N_PLANES = _N_TOKENS * _CODE_DIM
_P_PER_W = _N_PLANES // _NW
_LANES = 16


def _make_kernel():
    mesh = plsc.VectorSubcoreMesh(core_axis_name="c", subcore_axis_name="s")

    @functools.partial(
        pl.kernel,
        out_type=jax.ShapeDtypeStruct((_N_TOKENS, _CODE_DIM, _BATCH),
                                      jnp.float32),
        mesh=mesh,
        compiler_params=pltpu.CompilerParams(needs_layout_passes=False),
        scratch_types=[
            pltpu.VMEM((_N_SKELS,), jnp.float32),
            pltpu.VMEM((_HALF,), jnp.int32),
            pltpu.VMEM((_BATCH,), jnp.float32),
            pltpu.SemaphoreType.DMA,
            pltpu.SemaphoreType.DMA,
        ],
    )
    def k(idx_hbm, zm_hbm, off_hbm, out_hbm, plane_v, ids_v, out_v, sem,
          sem_out):
        wid = lax.axis_index("s") * _NC + lax.axis_index("c")
        pltpu.sync_copy(idx_hbm.at[pl.ds(0, _HALF)], ids_v)

        def plane_body(p, carry):
            pid = wid * _P_PER_W + p
            t = pid // _CODE_DIM
            c = pid % _CODE_DIM

            pltpu.sync_copy(zm_hbm.at[t, c], plane_v)
            @pl.when(p > 0)
            def _drain():
                tp = (pid - 1) // _CODE_DIM
                cp_ = (pid - 1) % _CODE_DIM
                pltpu.make_async_copy(out_v, out_hbm.at[tp, cp_],
                                      sem_out).wait()

            def zm_pass(h):
                @plsc.parallel_loop(0, _HALF, step=_LANES, unroll=16)
                def _(i):
                    ids16 = ids_v[pl.ds(i, _LANES)]
                    vals = plsc.load_gather(plane_v, [ids16])
                    out_v[pl.ds(h * _HALF + i, _LANES)] = vals

            zm_pass(0)
            pltpu.sync_copy(idx_hbm.at[pl.ds(_HALF, _HALF)], ids_v)
            zm_pass(1)

            pltpu.sync_copy(off_hbm.at[t, c], plane_v)

            def off_pass(h):
                @plsc.parallel_loop(0, _HALF, step=_LANES, unroll=16)
                def _(i):
                    ids16 = ids_v[pl.ds(i, _LANES)]
                    vals = plsc.load_gather(plane_v, [ids16])
                    mvec = jnp.where(ids16 < _N_TRAIN, jnp.float32(1.0),
                                     jnp.float32(0.0))
                    osl = pl.ds(h * _HALF + i, _LANES)
                    out_v[osl] = out_v[osl] + vals * mvec

            off_pass(1)
            pltpu.sync_copy(idx_hbm.at[pl.ds(0, _HALF)], ids_v)
            off_pass(0)

            pltpu.async_copy(out_v, out_hbm.at[t, c], sem_out)
            return carry

        lax.fori_loop(0, _P_PER_W, plane_body, 0, unroll=False)
        last = wid * _P_PER_W + _P_PER_W - 1
        pltpu.make_async_copy(
            out_v, out_hbm.at[last // _CODE_DIM, last % _CODE_DIM],
            sem_out).wait()

    return k


_kernel_call = _make_kernel()


@jax.jit
def kernel(tgt_skel_id, z_means, offset):
    zm_t = jnp.transpose(z_means, (1, 2, 0))
    off_t = jnp.transpose(offset, (1, 2, 0))
    out_t = _kernel_call(tgt_skel_id, zm_t, off_t)
    return jnp.transpose(out_t, (2, 0, 1))
